# Initial kernel scaffold; baseline (speedup 1.0000x reference)
import functools

import jax
import jax.numpy as jnp
from jax import lax
from jax.experimental import pallas as pl
from jax.experimental.pallas import tpu as pltpu
from jax.experimental.pallas import tpu_sc as plsc

NC, NS = 2, 16
NW = NC * NS
DEG = 8
EPS = 1e-5


def _mk_matvec(V, D):
    rpt = (2 * V) // NW
    C = min(rpt, max(16, 4096 // D))
    while rpt % C:
        C //= 2
    nch = rpt // C
    ng = (DEG * C) // 128
    mesh = plsc.VectorSubcoreMesh(core_axis_name="c", subcore_axis_name="s")

    @functools.partial(
        pl.kernel, mesh=mesh,
        out_type=jax.ShapeDtypeStruct((2 * V, D), jnp.float32),
        scratch_types=[
            pltpu.VMEM((DEG * C,), jnp.int32),
            pltpu.VMEM((ng, 128), jnp.int32),
            pltpu.VMEM((DEG * C,), jnp.float32),
            pltpu.VMEM((DEG * C, D), jnp.float32),
            pltpu.VMEM((C, D), jnp.float32),
            pltpu.SemaphoreType.DMA,
        ],
    )
    def mv(x_hbm, cols_hbm, vals_hbm, out_hbm, colbuf, idxbuf, valbuf, G, Y,
           sem):
        wid = lax.axis_index("s") * NC + lax.axis_index("c")
        row0 = wid * rpt
        b = row0 // V
        v0 = row0 - b * V
        boff = b * V

        def chunk(ci, carry):
            vb = v0 + ci * C
            eb = vb * DEG
            pltpu.sync_copy(cols_hbm.at[pl.ds(eb, DEG * C)], colbuf)
            pltpu.sync_copy(vals_hbm.at[pl.ds(eb, DEG * C)], valbuf)

            def mkidx(g, c2):
                for j in range(8):
                    idxbuf[g, pl.ds(j * 16, 16)] = (
                        colbuf[pl.ds(g * 128 + j * 16, 16)] + boff)
                return c2

            lax.fori_loop(0, ng, mkidx, 0)

            copies = []
            for g in range(ng):
                copies.append(pltpu.async_copy(
                    x_hbm.at[idxbuf.at[g]],
                    G.at[pl.ds(g * 128, 128)], sem))
            for cp in copies:
                cp.wait()

            def vert(vi, c2):
                e0 = vi * DEG
                svals = [valbuf[e0 + d] for d in range(DEG)]
                for j in range(D // 16):
                    sl = pl.ds(j * 16, 16)
                    acc = G[e0, sl] * svals[0]
                    for d in range(1, DEG):
                        acc = acc + G[e0 + d, sl] * svals[d]
                    Y[vi, sl] = acc
                return c2

            lax.fori_loop(0, C, vert, 0)
            pltpu.sync_copy(Y, out_hbm.at[pl.ds(row0 + ci * C, C)])
            return carry

        lax.fori_loop(0, nch, chunk, 0)

    return mv


def _mk_mm(N, Fin, Fout, bm):
    grid = N // bm

    def body(x0, x1, x2, w, h, stats, a1, a2):
        i = pl.program_id(0)

        @pl.when(i == 0)
        def _():
            a1[...] = jnp.zeros_like(a1)
            a2[...] = jnp.zeros_like(a2)

        w_ = w[...]
        hv = (jnp.dot(x0[...], w_[0:Fin, :], preferred_element_type=jnp.float32)
              + jnp.dot(x1[...], w_[Fin:2 * Fin, :],
                        preferred_element_type=jnp.float32)
              + jnp.dot(x2[...], w_[2 * Fin:3 * Fin, :],
                        preferred_element_type=jnp.float32))
        h[...] = hv
        s = jnp.sum(hv, axis=0, keepdims=True)
        ss = jnp.sum(hv * hv, axis=0, keepdims=True)
        a1[...] += jnp.broadcast_to(s, (8, Fout))
        a2[...] += jnp.broadcast_to(ss, (8, Fout))

        @pl.when(i == grid - 1)
        def _():
            stats[0:8, :] = a1[...]
            stats[8:16, :] = a2[...]

    return pl.pallas_call(
        body,
        grid=(grid,),
        in_specs=[
            pl.BlockSpec((bm, Fin), lambda i: (i, 0)),
            pl.BlockSpec((bm, Fin), lambda i: (i, 0)),
            pl.BlockSpec((bm, Fin), lambda i: (i, 0)),
            pl.BlockSpec((3 * Fin, Fout), lambda i: (0, 0)),
        ],
        out_specs=[
            pl.BlockSpec((bm, Fout), lambda i: (i, 0)),
            pl.BlockSpec((16, Fout), lambda i: (0, 0)),
        ],
        out_shape=[
            jax.ShapeDtypeStruct((N, Fout), jnp.float32),
            jax.ShapeDtypeStruct((16, Fout), jnp.float32),
        ],
        scratch_shapes=[
            pltpu.VMEM((8, Fout), jnp.float32),
            pltpu.VMEM((8, Fout), jnp.float32),
        ],
    )


def _bn_coeffs(stats, g, b, N, F):
    mean = stats[0:1, :] * (1.0 / N)
    var = stats[8:9, :] * (1.0 / N) - mean * mean
    sc = g * lax.rsqrt(var + EPS)
    sh = b - mean * sc
    return sc, sh


def _mk_bn(N, F, bm):
    grid = N // bm

    def body(h, stats, g, b, y):
        sc, sh = _bn_coeffs(stats[...], g[...], b[...], N, F)
        y[...] = jnp.maximum(h[...] * sc + sh, 0.0)

    return pl.pallas_call(
        body,
        grid=(grid,),
        in_specs=[
            pl.BlockSpec((bm, F), lambda i: (i, 0)),
            pl.BlockSpec((16, F), lambda i: (0, 0)),
            pl.BlockSpec((1, F), lambda i: (0, 0)),
            pl.BlockSpec((1, F), lambda i: (0, 0)),
        ],
        out_specs=pl.BlockSpec((bm, F), lambda i: (i, 0)),
        out_shape=jax.ShapeDtypeStruct((N, F), jnp.float32),
    )


def _mk_bn_pool(N, F, bm4):
    N4 = N // 4
    grid = N4 // bm4

    def body(h3, stats, g, b, y3, p):
        sc, sh = _bn_coeffs(stats[...], g[...], b[...], N, F)
        y = jnp.maximum(h3[...] * sc.reshape(1, 1, F) + sh.reshape(1, 1, F),
                        0.0)
        y3[...] = y
        p[...] = jnp.max(y, axis=1)

    return pl.pallas_call(
        body,
        grid=(grid,),
        in_specs=[
            pl.BlockSpec((bm4, 4, F), lambda i: (i, 0, 0)),
            pl.BlockSpec((16, F), lambda i: (0, 0)),
            pl.BlockSpec((1, F), lambda i: (0, 0)),
            pl.BlockSpec((1, F), lambda i: (0, 0)),
        ],
        out_specs=[
            pl.BlockSpec((bm4, 4, F), lambda i: (i, 0, 0)),
            pl.BlockSpec((bm4, F), lambda i: (i, 0)),
        ],
        out_shape=[
            jax.ShapeDtypeStruct((N4, 4, F), jnp.float32),
            jax.ShapeDtypeStruct((N4, F), jnp.float32),
        ],
    )


def _fold_weights(W, Fin):
    W0, W1, W2 = W[0::3], W[1::3], W[2::3]
    return jnp.concatenate([W0 - W2, W1, 2.0 * W2], axis=0)


def _conv_bn(X, cols, vals, W, gamma, beta, V, Fin, Fout, pool):
    N = 2 * V
    mvk = _mk_matvec(V, Fin)
    X1 = mvk(X, cols, vals)
    X2 = mvk(X1, cols, vals)
    Wc = _fold_weights(W, Fin)
    h, stats = _mk_mm(N, Fin, Fout, 512)(X, X1, X2, Wc)
    g2d = gamma.reshape(1, Fout)
    b2d = beta.reshape(1, Fout)
    if pool:
        y3, p = _mk_bn_pool(N, Fout, 512)(
            h.reshape(N // 4, 4, Fout), stats, g2d, b2d)
        return y3.reshape(N, Fout), p
    y = _mk_bn(N, Fout, 512)(h, stats, g2d, b2d)
    return y, None


def kernel(x, rows0, cols0, vals0, rows1, cols1, vals1, rows2, cols2, vals2,
           W1a, g1a, b1a, W1b, g1b, b1b, W2, g2, b2, W3, g3, b3):
    V0, V1, V2 = 49152, 12288, 3072
    X0 = x.reshape(2 * V0, 16)
    a, _ = _conv_bn(X0, cols0, vals0, W1a, g1a, b1a, V0, 16, 32, False)
    out1, p1 = _conv_bn(a, cols0, vals0, W1b, g1b, b1b, V0, 32, 64, True)
    out2, p2 = _conv_bn(p1, cols1, vals1, W2, g2, b2, V1, 64, 128, True)
    out3, _ = _conv_bn(p2, cols2, vals2, W3, g3, b3, V2, 128, 256, False)
    return (out3.reshape(2, V2, 256), out2.reshape(2, V1, 128),
            out1.reshape(2, V0, 64))


# speedup vs baseline: 151.3220x; 151.3220x over previous
import functools

import jax
import jax.numpy as jnp
from jax import lax
from jax.experimental import pallas as pl
from jax.experimental.pallas import tpu as pltpu
from jax.experimental.pallas import tpu_sc as plsc

NC, NS = 2, 16
NW = NC * NS
DEG = 8
EPS = 1e-5


def _mk_matvec(V, D):
    rpt = (2 * V) // NW
    C = min(rpt, max(16, 4096 // D))
    while rpt % C:
        C //= 2
    nch = rpt // C
    ng = (DEG * C) // 128
    mesh = plsc.VectorSubcoreMesh(core_axis_name="c", subcore_axis_name="s")

    @functools.partial(
        pl.kernel, mesh=mesh,
        compiler_params=pltpu.CompilerParams(use_tc_tiling_on_sc=False),
        out_type=jax.ShapeDtypeStruct((2 * V, D), jnp.float32),
        scratch_types=[
            pltpu.VMEM((DEG * C,), jnp.int32),
            pltpu.VMEM((ng, 128), jnp.int32),
            pltpu.VMEM((DEG * C,), jnp.float32),
            pltpu.VMEM((DEG * C, D), jnp.float32),
            pltpu.VMEM((C, D), jnp.float32),
            pltpu.SemaphoreType.DMA,
        ],
    )
    def mv(x_hbm, cols_hbm, vals_hbm, out_hbm, colbuf, idxbuf, valbuf, G, Y,
           sem):
        wid = lax.axis_index("s") * NC + lax.axis_index("c")
        row0 = wid * rpt
        b = row0 // V
        v0 = row0 - b * V
        boff = b * V

        def chunk(ci, carry):
            vb = v0 + ci * C
            eb = vb * DEG
            pltpu.sync_copy(cols_hbm.at[pl.ds(eb, DEG * C)], colbuf)
            pltpu.sync_copy(vals_hbm.at[pl.ds(eb, DEG * C)], valbuf)

            def mkidx(g, c2):
                for j in range(8):
                    idxbuf[g, pl.ds(j * 16, 16)] = (
                        colbuf[pl.ds(g * 128 + j * 16, 16)] + boff)
                return c2

            lax.fori_loop(0, ng, mkidx, 0)

            copies = []
            for g in range(ng):
                copies.append(pltpu.async_copy(
                    x_hbm.at[idxbuf.at[g]],
                    G.at[pl.ds(g * 128, 128)], sem))
            for cp in copies:
                cp.wait()

            def vert2(vi, c2):
                e0 = vi * 16
                vv = valbuf[pl.ds(e0, 16)]
                for h in range(2):
                    base = e0 + h * DEG
                    for j in range(D // 16):
                        sl = pl.ds(j * 16, 16)
                        acc = G[base, sl] * vv[h * DEG]
                        for d in range(1, DEG):
                            acc = acc + G[base + d, sl] * vv[h * DEG + d]
                        Y[2 * vi + h, sl] = acc
                return c2

            lax.fori_loop(0, C // 2, vert2, 0)
            pltpu.sync_copy(Y, out_hbm.at[pl.ds(row0 + ci * C, C)])
            return carry

        lax.fori_loop(0, nch, chunk, 0)

    return mv


def _mk_mm(N, Fin, Fout, bm):
    grid = N // bm

    def body(x0, x1, x2, w, h, stats, a1, a2):
        i = pl.program_id(0)

        @pl.when(i == 0)
        def _():
            a1[...] = jnp.zeros_like(a1)
            a2[...] = jnp.zeros_like(a2)

        w_ = w[...]
        x0v = x0[...]
        x2c = 2.0 * x2[...] - x0v
        hv = (jnp.dot(x0v, w_[0:Fin, :], preferred_element_type=jnp.float32)
              + jnp.dot(x1[...], w_[Fin:2 * Fin, :],
                        preferred_element_type=jnp.float32)
              + jnp.dot(x2c, w_[2 * Fin:3 * Fin, :],
                        preferred_element_type=jnp.float32))
        h[...] = hv
        s = jnp.sum(hv, axis=0, keepdims=True)
        ss = jnp.sum(hv * hv, axis=0, keepdims=True)
        a1[...] += jnp.broadcast_to(s, (8, Fout))
        a2[...] += jnp.broadcast_to(ss, (8, Fout))

        @pl.when(i == grid - 1)
        def _():
            stats[0:8, :] = a1[...]
            stats[8:16, :] = a2[...]

    return pl.pallas_call(
        body,
        grid=(grid,),
        in_specs=[
            pl.BlockSpec((bm, Fin), lambda i: (i, 0)),
            pl.BlockSpec((bm, Fin), lambda i: (i, 0)),
            pl.BlockSpec((bm, Fin), lambda i: (i, 0)),
            pl.BlockSpec((3 * Fin, Fout), lambda i: (0, 0)),
        ],
        out_specs=[
            pl.BlockSpec((bm, Fout), lambda i: (i, 0)),
            pl.BlockSpec((16, Fout), lambda i: (0, 0)),
        ],
        out_shape=[
            jax.ShapeDtypeStruct((N, Fout), jnp.float32),
            jax.ShapeDtypeStruct((16, Fout), jnp.float32),
        ],
        scratch_shapes=[
            pltpu.VMEM((8, Fout), jnp.float32),
            pltpu.VMEM((8, Fout), jnp.float32),
        ],
    )


def _bn_coeffs(stats, g, b, N, F):
    mean = stats[0:1, :] * (1.0 / N)
    var = stats[8:9, :] * (1.0 / N) - mean * mean
    sc = g * lax.rsqrt(var + EPS)
    sh = b - mean * sc
    return sc, sh


def _mk_bn(N, F, bm):
    grid = N // bm

    def body(h, stats, g, b, y):
        sc, sh = _bn_coeffs(stats[...], g[...], b[...], N, F)
        y[...] = jnp.maximum(h[...] * sc + sh, 0.0)

    return pl.pallas_call(
        body,
        grid=(grid,),
        in_specs=[
            pl.BlockSpec((bm, F), lambda i: (i, 0)),
            pl.BlockSpec((16, F), lambda i: (0, 0)),
            pl.BlockSpec((1, F), lambda i: (0, 0)),
            pl.BlockSpec((1, F), lambda i: (0, 0)),
        ],
        out_specs=pl.BlockSpec((bm, F), lambda i: (i, 0)),
        out_shape=jax.ShapeDtypeStruct((N, F), jnp.float32),
    )


def _mk_bn_pool(N, F, bm4):
    N4 = N // 4
    grid = N4 // bm4

    def body(h3, stats, g, b, y3, p):
        sc, sh = _bn_coeffs(stats[...], g[...], b[...], N, F)
        y = jnp.maximum(h3[...] * sc.reshape(1, 1, F) + sh.reshape(1, 1, F),
                        0.0)
        y3[...] = y
        p[...] = jnp.max(y, axis=1)

    return pl.pallas_call(
        body,
        grid=(grid,),
        in_specs=[
            pl.BlockSpec((bm4, 4, F), lambda i: (i, 0, 0)),
            pl.BlockSpec((16, F), lambda i: (0, 0)),
            pl.BlockSpec((1, F), lambda i: (0, 0)),
            pl.BlockSpec((1, F), lambda i: (0, 0)),
        ],
        out_specs=[
            pl.BlockSpec((bm4, 4, F), lambda i: (i, 0, 0)),
            pl.BlockSpec((bm4, F), lambda i: (i, 0)),
        ],
        out_shape=[
            jax.ShapeDtypeStruct((N4, 4, F), jnp.float32),
            jax.ShapeDtypeStruct((N4, F), jnp.float32),
        ],
    )


def _fold_weights(W, Fin):
    W0, W1, W2 = W[0::3], W[1::3], W[2::3]
    return jnp.concatenate([W0, W1, W2], axis=0)


def _conv_bn(X, cols, vals, W, gamma, beta, V, Fin, Fout, pool):
    N = 2 * V
    mvk = _mk_matvec(V, Fin)
    X1 = mvk(X, cols, vals)
    X2 = mvk(X1, cols, vals)
    Wc = _fold_weights(W, Fin)
    h, stats = _mk_mm(N, Fin, Fout, 512)(X, X1, X2, Wc)
    g2d = gamma.reshape(1, Fout)
    b2d = beta.reshape(1, Fout)
    if pool:
        y3, p = _mk_bn_pool(N, Fout, 512)(
            h.reshape(N // 4, 4, Fout), stats, g2d, b2d)
        return y3.reshape(N, Fout), p
    y = _mk_bn(N, Fout, 512)(h, stats, g2d, b2d)
    return y, None


def kernel(x, rows0, cols0, vals0, rows1, cols1, vals1, rows2, cols2, vals2,
           W1a, g1a, b1a, W1b, g1b, b1b, W2, g2, b2, W3, g3, b3):
    V0, V1, V2 = 49152, 12288, 3072
    X0 = x.reshape(2 * V0, 16)
    a, _ = _conv_bn(X0, cols0, vals0, W1a, g1a, b1a, V0, 16, 32, False)
    out1, p1 = _conv_bn(a, cols0, vals0, W1b, g1b, b1b, V0, 32, 64, True)
    out2, p2 = _conv_bn(p1, cols1, vals1, W2, g2, b2, V1, 64, 128, True)
    out3, _ = _conv_bn(p2, cols2, vals2, W3, g3, b3, V2, 128, 256, False)
    return (out3.reshape(2, V2, 256), out2.reshape(2, V1, 128),
            out1.reshape(2, V0, 64))


# speedup vs baseline: 173.3592x; 1.1456x over previous
import functools

import jax
import jax.numpy as jnp
from jax import lax
from jax.experimental import pallas as pl
from jax.experimental.pallas import tpu as pltpu
from jax.experimental.pallas import tpu_sc as plsc

NC, NS = 2, 16
NW = NC * NS
DEG = 8
EPS = 1e-5


def _mk_matvec(V, D):
    rpt = (2 * V) // NW
    C = min(rpt, max(16, 4096 // D))
    while rpt % C:
        C //= 2
    nch = rpt // C
    ng = (DEG * C) // 128
    mesh = plsc.VectorSubcoreMesh(core_axis_name="c", subcore_axis_name="s")

    @functools.partial(
        pl.kernel, mesh=mesh,
        compiler_params=pltpu.CompilerParams(use_tc_tiling_on_sc=False),
        out_type=jax.ShapeDtypeStruct((2 * V, D), jnp.float32),
        scratch_types=[
            pltpu.VMEM((2, DEG * C), jnp.int32),
            pltpu.VMEM((2, ng, 128), jnp.int32),
            pltpu.VMEM((2, DEG * C), jnp.float32),
            pltpu.VMEM((2, DEG * C, D), jnp.float32),
            pltpu.VMEM((C, D), jnp.float32),
            pltpu.SemaphoreType.DMA,
        ],
    )
    def mv(x_hbm, cols_hbm, vals_hbm, out_hbm, colbuf, idxbuf, valbuf, G, Y,
           sem):
        wid = lax.axis_index("s") * NC + lax.axis_index("c")
        row0 = wid * rpt
        b = row0 // V
        v0 = row0 - b * V
        boff = b * V

        def stage(ci, buf):
            eb = (v0 + ci * C) * DEG
            pltpu.sync_copy(cols_hbm.at[pl.ds(eb, DEG * C)], colbuf.at[buf])
            pltpu.sync_copy(vals_hbm.at[pl.ds(eb, DEG * C)], valbuf.at[buf])

            def mkidx(g, c2):
                for j in range(8):
                    idxbuf[buf, g, pl.ds(j * 16, 16)] = (
                        colbuf[buf, pl.ds(g * 128 + j * 16, 16)] + boff)
                return c2

            lax.fori_loop(0, ng, mkidx, 0, unroll=2)
            for g in range(ng):
                pltpu.async_copy(
                    x_hbm.at[idxbuf.at[buf].at[g]],
                    G.at[buf].at[pl.ds(g * 128, 128)], sem)

        def compute(ci, buf):
            pltpu.make_async_copy(
                x_hbm.at[pl.ds(0, DEG * C)], G.at[buf], sem).wait()

            def vert2(vi, c2):
                e0 = vi * 16
                vv = valbuf[buf, pl.ds(e0, 16)]
                for h in range(2):
                    base = e0 + h * DEG
                    for j in range(D // 16):
                        sl = pl.ds(j * 16, 16)
                        acc = G[buf, base, sl] * vv[h * DEG]
                        for d in range(1, DEG):
                            acc = acc + G[buf, base + d, sl] * vv[h * DEG + d]
                        Y[2 * vi + h, sl] = acc
                return c2

            lax.fori_loop(0, C // 2, vert2, 0, unroll=2)
            pltpu.sync_copy(Y, out_hbm.at[pl.ds(row0 + ci * C, C)])

        stage(0, 0)

        def pair(p, carry):
            n0 = 2 * p
            for off in range(2):
                n = n0 + off
                buf = off

                @pl.when(n + 1 < nch)
                def _():
                    stage(n + 1, 1 - buf)

                compute(n, buf)
            return carry

        lax.fori_loop(0, nch // 2, pair, 0)

    return mv


def _mk_mm(N, Fin, Fout, bm):
    grid = N // bm

    def body(x0, x1, x2, w, h, stats, a1, a2):
        i = pl.program_id(0)

        @pl.when(i == 0)
        def _():
            a1[...] = jnp.zeros_like(a1)
            a2[...] = jnp.zeros_like(a2)

        w_ = w[...]
        x0v = x0[...]
        x2c = 2.0 * x2[...] - x0v
        hv = (jnp.dot(x0v, w_[0:Fin, :], preferred_element_type=jnp.float32)
              + jnp.dot(x1[...], w_[Fin:2 * Fin, :],
                        preferred_element_type=jnp.float32)
              + jnp.dot(x2c, w_[2 * Fin:3 * Fin, :],
                        preferred_element_type=jnp.float32))
        h[...] = hv
        s = jnp.sum(hv, axis=0, keepdims=True)
        ss = jnp.sum(hv * hv, axis=0, keepdims=True)
        a1[...] += jnp.broadcast_to(s, (8, Fout))
        a2[...] += jnp.broadcast_to(ss, (8, Fout))

        @pl.when(i == grid - 1)
        def _():
            stats[0:8, :] = a1[...]
            stats[8:16, :] = a2[...]

    return pl.pallas_call(
        body,
        grid=(grid,),
        in_specs=[
            pl.BlockSpec((bm, Fin), lambda i: (i, 0)),
            pl.BlockSpec((bm, Fin), lambda i: (i, 0)),
            pl.BlockSpec((bm, Fin), lambda i: (i, 0)),
            pl.BlockSpec((3 * Fin, Fout), lambda i: (0, 0)),
        ],
        out_specs=[
            pl.BlockSpec((bm, Fout), lambda i: (i, 0)),
            pl.BlockSpec((16, Fout), lambda i: (0, 0)),
        ],
        out_shape=[
            jax.ShapeDtypeStruct((N, Fout), jnp.float32),
            jax.ShapeDtypeStruct((16, Fout), jnp.float32),
        ],
        scratch_shapes=[
            pltpu.VMEM((8, Fout), jnp.float32),
            pltpu.VMEM((8, Fout), jnp.float32),
        ],
    )


def _bn_coeffs(stats, g, b, N, F):
    mean = stats[0:1, :] * (1.0 / N)
    var = stats[8:9, :] * (1.0 / N) - mean * mean
    sc = g * lax.rsqrt(var + EPS)
    sh = b - mean * sc
    return sc, sh


def _mk_bn(N, F, bm):
    grid = N // bm

    def body(h, stats, g, b, y):
        sc, sh = _bn_coeffs(stats[...], g[...], b[...], N, F)
        y[...] = jnp.maximum(h[...] * sc + sh, 0.0)

    return pl.pallas_call(
        body,
        grid=(grid,),
        in_specs=[
            pl.BlockSpec((bm, F), lambda i: (i, 0)),
            pl.BlockSpec((16, F), lambda i: (0, 0)),
            pl.BlockSpec((1, F), lambda i: (0, 0)),
            pl.BlockSpec((1, F), lambda i: (0, 0)),
        ],
        out_specs=pl.BlockSpec((bm, F), lambda i: (i, 0)),
        out_shape=jax.ShapeDtypeStruct((N, F), jnp.float32),
    )


def _mk_bn_pool(N, F, bm4):
    N4 = N // 4
    grid = N4 // bm4

    def body(h3, stats, g, b, y3, p):
        sc, sh = _bn_coeffs(stats[...], g[...], b[...], N, F)
        y = jnp.maximum(h3[...] * sc.reshape(1, 1, F) + sh.reshape(1, 1, F),
                        0.0)
        y3[...] = y
        p[...] = jnp.max(y, axis=1)

    return pl.pallas_call(
        body,
        grid=(grid,),
        in_specs=[
            pl.BlockSpec((bm4, 4, F), lambda i: (i, 0, 0)),
            pl.BlockSpec((16, F), lambda i: (0, 0)),
            pl.BlockSpec((1, F), lambda i: (0, 0)),
            pl.BlockSpec((1, F), lambda i: (0, 0)),
        ],
        out_specs=[
            pl.BlockSpec((bm4, 4, F), lambda i: (i, 0, 0)),
            pl.BlockSpec((bm4, F), lambda i: (i, 0)),
        ],
        out_shape=[
            jax.ShapeDtypeStruct((N4, 4, F), jnp.float32),
            jax.ShapeDtypeStruct((N4, F), jnp.float32),
        ],
    )


def _fold_weights(W, Fin):
    W0, W1, W2 = W[0::3], W[1::3], W[2::3]
    return jnp.concatenate([W0, W1, W2], axis=0)


def _conv_bn(X, cols, vals, W, gamma, beta, V, Fin, Fout, pool):
    N = 2 * V
    mvk = _mk_matvec(V, Fin)
    X1 = mvk(X, cols, vals)
    X2 = mvk(X1, cols, vals)
    Wc = _fold_weights(W, Fin)
    h, stats = _mk_mm(N, Fin, Fout, 512)(X, X1, X2, Wc)
    g2d = gamma.reshape(1, Fout)
    b2d = beta.reshape(1, Fout)
    if pool:
        y3, p = _mk_bn_pool(N, Fout, 512)(
            h.reshape(N // 4, 4, Fout), stats, g2d, b2d)
        return y3.reshape(N, Fout), p
    y = _mk_bn(N, Fout, 512)(h, stats, g2d, b2d)
    return y, None


def kernel(x, rows0, cols0, vals0, rows1, cols1, vals1, rows2, cols2, vals2,
           W1a, g1a, b1a, W1b, g1b, b1b, W2, g2, b2, W3, g3, b3):
    V0, V1, V2 = 49152, 12288, 3072
    X0 = x.reshape(2 * V0, 16)
    a, _ = _conv_bn(X0, cols0, vals0, W1a, g1a, b1a, V0, 16, 32, False)
    out1, p1 = _conv_bn(a, cols0, vals0, W1b, g1b, b1b, V0, 32, 64, True)
    out2, p2 = _conv_bn(p1, cols1, vals1, W2, g2, b2, V1, 64, 128, True)
    out3, _ = _conv_bn(p2, cols2, vals2, W3, g3, b3, V2, 128, 256, False)
    return (out3.reshape(2, V2, 256), out2.reshape(2, V1, 128),
            out1.reshape(2, V0, 64))


# speedup vs baseline: 213.8630x; 1.2336x over previous
import functools

import jax
import jax.numpy as jnp
from jax import lax
from jax.experimental import pallas as pl
from jax.experimental.pallas import tpu as pltpu
from jax.experimental.pallas import tpu_sc as plsc

NC, NS = 2, 16
NW = NC * NS
DEG = 8
EPS = 1e-5


def _mk_matvec(V, D):
    rpt = (2 * V) // NW
    C = min(rpt, max(16, 4096 // D))
    while rpt % C:
        C //= 2
    nch = rpt // C
    ng = (DEG * C) // 128
    mesh = plsc.VectorSubcoreMesh(core_axis_name="c", subcore_axis_name="s")

    @functools.partial(
        pl.kernel, mesh=mesh,
        compiler_params=pltpu.CompilerParams(use_tc_tiling_on_sc=False),
        out_type=jax.ShapeDtypeStruct((2 * V, D), jnp.float32),
        scratch_types=[
            pltpu.VMEM((2, DEG * C), jnp.int32),
            pltpu.VMEM((2, ng, 128), jnp.int32),
            pltpu.VMEM((2, DEG * C), jnp.float32),
            pltpu.VMEM((2, DEG * C, D), jnp.float32),
            pltpu.VMEM((C, D), jnp.float32),
            pltpu.SemaphoreType.DMA,
        ],
    )
    def mv(x_hbm, cols_hbm, vals_hbm, out_hbm, colbuf, idxbuf, valbuf, G, Y,
           sem):
        wid = lax.axis_index("s") * NC + lax.axis_index("c")
        row0 = wid * rpt
        b = row0 // V
        v0 = row0 - b * V
        boff = b * V

        def stage(ci, buf):
            eb = (v0 + ci * C) * DEG
            pltpu.sync_copy(cols_hbm.at[pl.ds(eb, DEG * C)], colbuf.at[buf])
            pltpu.sync_copy(vals_hbm.at[pl.ds(eb, DEG * C)], valbuf.at[buf])

            def mkidx(g, c2):
                for j in range(8):
                    idxbuf[buf, g, pl.ds(j * 16, 16)] = (
                        colbuf[buf, pl.ds(g * 128 + j * 16, 16)] + boff)
                return c2

            lax.fori_loop(0, ng, mkidx, 0, unroll=2)
            for g in range(ng):
                pltpu.async_copy(
                    x_hbm.at[idxbuf.at[buf].at[g]],
                    G.at[buf].at[pl.ds(g * 128, 128)], sem)

        def compute(ci, buf):
            pltpu.make_async_copy(
                x_hbm.at[pl.ds(0, DEG * C)], G.at[buf], sem).wait()

            def vert2(vi, c2):
                e0 = vi * 16
                vv = valbuf[buf, pl.ds(e0, 16)]
                for h in range(2):
                    base = e0 + h * DEG
                    for j in range(D // 16):
                        sl = pl.ds(j * 16, 16)
                        acc = G[buf, base, sl] * vv[h * DEG]
                        for d in range(1, DEG):
                            acc = acc + G[buf, base + d, sl] * vv[h * DEG + d]
                        Y[2 * vi + h, sl] = acc
                return c2

            lax.fori_loop(0, C // 2, vert2, 0, unroll=2)
            pltpu.sync_copy(Y, out_hbm.at[pl.ds(row0 + ci * C, C)])

        stage(0, 0)

        def pair(p, carry):
            n0 = 2 * p
            for off in range(2):
                n = n0 + off
                buf = off

                @pl.when(n + 1 < nch)
                def _():
                    stage(n + 1, 1 - buf)

                compute(n, buf)
            return carry

        lax.fori_loop(0, nch // 2, pair, 0)

    return mv


def _mk_conv_fused(N, Fin, Fout, bm, pool):
    nb = N // bm

    def body(x0, x1, x2, w, g, b, *rest):
        if pool:
            y, p, a1, a2 = rest
        else:
            (y, a1, a2) = rest
        ph = pl.program_id(0)
        i = pl.program_id(1)

        @pl.when(jnp.logical_and(ph == 0, i == 0))
        def _():
            a1[...] = jnp.zeros_like(a1)
            a2[...] = jnp.zeros_like(a2)

        w_ = w[...]
        x0v = x0[...]
        x2c = 2.0 * x2[...] - x0v
        hv = (jnp.dot(x0v, w_[0:Fin, :], preferred_element_type=jnp.float32)
              + jnp.dot(x1[...], w_[Fin:2 * Fin, :],
                        preferred_element_type=jnp.float32)
              + jnp.dot(x2c, w_[2 * Fin:3 * Fin, :],
                        preferred_element_type=jnp.float32))

        @pl.when(ph == 0)
        def _():
            s = jnp.sum(hv, axis=0, keepdims=True)
            ss = jnp.sum(hv * hv, axis=0, keepdims=True)
            a1[...] += jnp.broadcast_to(s, (8, Fout))
            a2[...] += jnp.broadcast_to(ss, (8, Fout))

        @pl.when(ph == 1)
        def _():
            mean = a1[0:1, :] * (1.0 / N)
            var = a2[0:1, :] * (1.0 / N) - mean * mean
            sc = g[...] * lax.rsqrt(var + EPS)
            sh = b[...] - mean * sc
            yv = jnp.maximum(hv * sc + sh, 0.0)
            y[...] = yv
            if pool:
                p[...] = yv.reshape(bm // 4, 4, Fout).max(axis=1)

    out_specs = [pl.BlockSpec((bm, Fout), lambda ph, i: (ph * i, 0))]
    out_shape = [jax.ShapeDtypeStruct((N, Fout), jnp.float32)]
    if pool:
        out_specs.append(pl.BlockSpec((bm // 4, Fout), lambda ph, i: (ph * i, 0)))
        out_shape.append(jax.ShapeDtypeStruct((N // 4, Fout), jnp.float32))

    return pl.pallas_call(
        body,
        grid=(2, nb),
        in_specs=[
            pl.BlockSpec((bm, Fin), lambda ph, i: (i, 0)),
            pl.BlockSpec((bm, Fin), lambda ph, i: (i, 0)),
            pl.BlockSpec((bm, Fin), lambda ph, i: (i, 0)),
            pl.BlockSpec((3 * Fin, Fout), lambda ph, i: (0, 0)),
            pl.BlockSpec((1, Fout), lambda ph, i: (0, 0)),
            pl.BlockSpec((1, Fout), lambda ph, i: (0, 0)),
        ],
        out_specs=out_specs,
        out_shape=out_shape,
        scratch_shapes=[
            pltpu.VMEM((8, Fout), jnp.float32),
            pltpu.VMEM((8, Fout), jnp.float32),
        ],
    )


def _mk_mm(N, Fin, Fout, bm):
    grid = N // bm

    def body(x0, x1, x2, w, h, stats, a1, a2):
        i = pl.program_id(0)

        @pl.when(i == 0)
        def _():
            a1[...] = jnp.zeros_like(a1)
            a2[...] = jnp.zeros_like(a2)

        w_ = w[...]
        x0v = x0[...]
        x2c = 2.0 * x2[...] - x0v
        hv = (jnp.dot(x0v, w_[0:Fin, :], preferred_element_type=jnp.float32)
              + jnp.dot(x1[...], w_[Fin:2 * Fin, :],
                        preferred_element_type=jnp.float32)
              + jnp.dot(x2c, w_[2 * Fin:3 * Fin, :],
                        preferred_element_type=jnp.float32))
        h[...] = hv
        s = jnp.sum(hv, axis=0, keepdims=True)
        ss = jnp.sum(hv * hv, axis=0, keepdims=True)
        a1[...] += jnp.broadcast_to(s, (8, Fout))
        a2[...] += jnp.broadcast_to(ss, (8, Fout))

        @pl.when(i == grid - 1)
        def _():
            stats[0:8, :] = a1[...]
            stats[8:16, :] = a2[...]

    return pl.pallas_call(
        body,
        grid=(grid,),
        in_specs=[
            pl.BlockSpec((bm, Fin), lambda i: (i, 0)),
            pl.BlockSpec((bm, Fin), lambda i: (i, 0)),
            pl.BlockSpec((bm, Fin), lambda i: (i, 0)),
            pl.BlockSpec((3 * Fin, Fout), lambda i: (0, 0)),
        ],
        out_specs=[
            pl.BlockSpec((bm, Fout), lambda i: (i, 0)),
            pl.BlockSpec((16, Fout), lambda i: (0, 0)),
        ],
        out_shape=[
            jax.ShapeDtypeStruct((N, Fout), jnp.float32),
            jax.ShapeDtypeStruct((16, Fout), jnp.float32),
        ],
        scratch_shapes=[
            pltpu.VMEM((8, Fout), jnp.float32),
            pltpu.VMEM((8, Fout), jnp.float32),
        ],
    )


def _bn_coeffs(stats, g, b, N, F):
    mean = stats[0:1, :] * (1.0 / N)
    var = stats[8:9, :] * (1.0 / N) - mean * mean
    sc = g * lax.rsqrt(var + EPS)
    sh = b - mean * sc
    return sc, sh


def _mk_bn(N, F, bm):
    grid = N // bm

    def body(h, stats, g, b, y):
        sc, sh = _bn_coeffs(stats[...], g[...], b[...], N, F)
        y[...] = jnp.maximum(h[...] * sc + sh, 0.0)

    return pl.pallas_call(
        body,
        grid=(grid,),
        in_specs=[
            pl.BlockSpec((bm, F), lambda i: (i, 0)),
            pl.BlockSpec((16, F), lambda i: (0, 0)),
            pl.BlockSpec((1, F), lambda i: (0, 0)),
            pl.BlockSpec((1, F), lambda i: (0, 0)),
        ],
        out_specs=pl.BlockSpec((bm, F), lambda i: (i, 0)),
        out_shape=jax.ShapeDtypeStruct((N, F), jnp.float32),
    )


def _mk_bn_pool(N, F, bm4):
    N4 = N // 4
    grid = N4 // bm4

    def body(h3, stats, g, b, y3, p):
        sc, sh = _bn_coeffs(stats[...], g[...], b[...], N, F)
        y = jnp.maximum(h3[...] * sc.reshape(1, 1, F) + sh.reshape(1, 1, F),
                        0.0)
        y3[...] = y
        p[...] = jnp.max(y, axis=1)

    return pl.pallas_call(
        body,
        grid=(grid,),
        in_specs=[
            pl.BlockSpec((bm4, 4, F), lambda i: (i, 0, 0)),
            pl.BlockSpec((16, F), lambda i: (0, 0)),
            pl.BlockSpec((1, F), lambda i: (0, 0)),
            pl.BlockSpec((1, F), lambda i: (0, 0)),
        ],
        out_specs=[
            pl.BlockSpec((bm4, 4, F), lambda i: (i, 0, 0)),
            pl.BlockSpec((bm4, F), lambda i: (i, 0)),
        ],
        out_shape=[
            jax.ShapeDtypeStruct((N4, 4, F), jnp.float32),
            jax.ShapeDtypeStruct((N4, F), jnp.float32),
        ],
    )


def _fold_weights(W, Fin):
    W0, W1, W2 = W[0::3], W[1::3], W[2::3]
    return jnp.concatenate([W0, W1, W2], axis=0)


def _conv_bn(X, cols, vals, W, gamma, beta, V, Fin, Fout, pool):
    N = 2 * V
    mvk = _mk_matvec(V, Fin)
    X1 = mvk(X, cols, vals)
    X2 = mvk(X1, cols, vals)
    Wc = _fold_weights(W, Fin)
    bm = {98304: 4096, 24576: 2048, 6144: 1024}[N]
    outs = _mk_conv_fused(N, Fin, Fout, bm, pool)(
        X, X1, X2, Wc, gamma.reshape(1, Fout), beta.reshape(1, Fout))
    if pool:
        return outs[0], outs[1]
    return outs[0], None


def kernel(x, rows0, cols0, vals0, rows1, cols1, vals1, rows2, cols2, vals2,
           W1a, g1a, b1a, W1b, g1b, b1b, W2, g2, b2, W3, g3, b3):
    V0, V1, V2 = 49152, 12288, 3072
    X0 = x.reshape(2 * V0, 16)
    a, _ = _conv_bn(X0, cols0, vals0, W1a, g1a, b1a, V0, 16, 32, False)
    out1, p1 = _conv_bn(a, cols0, vals0, W1b, g1b, b1b, V0, 32, 64, True)
    out2, p2 = _conv_bn(p1, cols1, vals1, W2, g2, b2, V1, 64, 128, True)
    out3, _ = _conv_bn(p2, cols2, vals2, W3, g3, b3, V2, 128, 256, False)
    return (out3.reshape(2, V2, 256), out2.reshape(2, V1, 128),
            out1.reshape(2, V0, 64))


# speedup vs baseline: 215.1975x; 1.0062x over previous
import functools

import jax
import jax.numpy as jnp
from jax import lax
from jax.experimental import pallas as pl
from jax.experimental.pallas import tpu as pltpu
from jax.experimental.pallas import tpu_sc as plsc

NC, NS = 2, 16
NW = NC * NS
DEG = 8
EPS = 1e-5


def _mk_matvec(V, D):
    rpt = (2 * V) // NW
    C = min(rpt, max(16, 4096 // D))
    while rpt % C:
        C //= 2
    nch = rpt // C
    ng = (DEG * C) // 128
    mesh = plsc.VectorSubcoreMesh(core_axis_name="c", subcore_axis_name="s")

    @functools.partial(
        pl.kernel, mesh=mesh,
        compiler_params=pltpu.CompilerParams(use_tc_tiling_on_sc=False),
        out_type=jax.ShapeDtypeStruct((2 * V, D), jnp.float32),
        scratch_types=[
            pltpu.VMEM((2, DEG * C), jnp.int32),
            pltpu.VMEM((2, ng, 128), jnp.int32),
            pltpu.VMEM((2, DEG * C), jnp.float32),
            pltpu.VMEM((2, DEG * C, D), jnp.float32),
            pltpu.VMEM((C, D), jnp.float32),
            pltpu.SemaphoreType.DMA,
        ],
    )
    def mv(x_hbm, cols_hbm, vals_hbm, out_hbm, colbuf, idxbuf, valbuf, G, Y,
           sem):
        wid = lax.axis_index("s") * NC + lax.axis_index("c")
        row0 = wid * rpt
        b = row0 // V
        v0 = row0 - b * V
        boff = b * V

        def stage(ci, buf):
            eb = (v0 + ci * C) * DEG
            pltpu.sync_copy(cols_hbm.at[pl.ds(eb, DEG * C)], colbuf.at[buf])
            pltpu.sync_copy(vals_hbm.at[pl.ds(eb, DEG * C)], valbuf.at[buf])

            def mkidx(g, c2):
                for j in range(8):
                    idxbuf[buf, g, pl.ds(j * 16, 16)] = (
                        colbuf[buf, pl.ds(g * 128 + j * 16, 16)] + boff)
                return c2

            lax.fori_loop(0, ng, mkidx, 0, unroll=2)
            for g in range(ng):
                pltpu.async_copy(
                    x_hbm.at[idxbuf.at[buf].at[g]],
                    G.at[buf].at[pl.ds(g * 128, 128)], sem)

        def compute(ci, buf):
            pltpu.make_async_copy(
                x_hbm.at[pl.ds(0, DEG * C)], G.at[buf], sem).wait()

            def vert2(vi, c2):
                e0 = vi * 16
                vv = valbuf[buf, pl.ds(e0, 16)]
                for h in range(2):
                    base = e0 + h * DEG
                    for j in range(D // 16):
                        sl = pl.ds(j * 16, 16)
                        acc = G[buf, base, sl] * vv[h * DEG]
                        for d in range(1, DEG):
                            acc = acc + G[buf, base + d, sl] * vv[h * DEG + d]
                        Y[2 * vi + h, sl] = acc
                return c2

            lax.fori_loop(0, C // 2, vert2, 0, unroll=4)
            pltpu.sync_copy(Y, out_hbm.at[pl.ds(row0 + ci * C, C)])

        stage(0, 0)

        def pair(p, carry):
            n0 = 2 * p
            for off in range(2):
                n = n0 + off
                buf = off

                @pl.when(n + 1 < nch)
                def _():
                    stage(n + 1, 1 - buf)

                compute(n, buf)
            return carry

        lax.fori_loop(0, nch // 2, pair, 0)

    return mv


def _mk_conv_fused(N, Fin, Fout, bm, pool):
    nb = N // bm

    def body(x0, x1, x2, w, g, b, *rest):
        if pool:
            y, p, a1, a2 = rest
        else:
            (y, a1, a2) = rest
        ph = pl.program_id(0)
        i = pl.program_id(1)

        @pl.when(jnp.logical_and(ph == 0, i == 0))
        def _():
            a1[...] = jnp.zeros_like(a1)
            a2[...] = jnp.zeros_like(a2)

        w_ = w[...]
        x0v = x0[...]
        x2c = 2.0 * x2[...] - x0v
        hv = (jnp.dot(x0v, w_[0:Fin, :], preferred_element_type=jnp.float32)
              + jnp.dot(x1[...], w_[Fin:2 * Fin, :],
                        preferred_element_type=jnp.float32)
              + jnp.dot(x2c, w_[2 * Fin:3 * Fin, :],
                        preferred_element_type=jnp.float32))

        @pl.when(ph == 0)
        def _():
            s = jnp.sum(hv, axis=0, keepdims=True)
            ss = jnp.sum(hv * hv, axis=0, keepdims=True)
            a1[...] += jnp.broadcast_to(s, (8, Fout))
            a2[...] += jnp.broadcast_to(ss, (8, Fout))

        @pl.when(ph == 1)
        def _():
            mean = a1[0:1, :] * (1.0 / N)
            var = a2[0:1, :] * (1.0 / N) - mean * mean
            sc = g[...] * lax.rsqrt(var + EPS)
            sh = b[...] - mean * sc
            yv = jnp.maximum(hv * sc + sh, 0.0)
            y[...] = yv
            if pool:
                p[...] = yv.reshape(bm // 4, 4, Fout).max(axis=1)

    out_specs = [pl.BlockSpec((bm, Fout), lambda ph, i: (ph * i, 0))]
    out_shape = [jax.ShapeDtypeStruct((N, Fout), jnp.float32)]
    if pool:
        out_specs.append(pl.BlockSpec((bm // 4, Fout), lambda ph, i: (ph * i, 0)))
        out_shape.append(jax.ShapeDtypeStruct((N // 4, Fout), jnp.float32))

    return pl.pallas_call(
        body,
        grid=(2, nb),
        in_specs=[
            pl.BlockSpec((bm, Fin), lambda ph, i: (i, 0)),
            pl.BlockSpec((bm, Fin), lambda ph, i: (i, 0)),
            pl.BlockSpec((bm, Fin), lambda ph, i: (i, 0)),
            pl.BlockSpec((3 * Fin, Fout), lambda ph, i: (0, 0)),
            pl.BlockSpec((1, Fout), lambda ph, i: (0, 0)),
            pl.BlockSpec((1, Fout), lambda ph, i: (0, 0)),
        ],
        out_specs=out_specs,
        out_shape=out_shape,
        scratch_shapes=[
            pltpu.VMEM((8, Fout), jnp.float32),
            pltpu.VMEM((8, Fout), jnp.float32),
        ],
    )


def _mk_mm(N, Fin, Fout, bm):
    grid = N // bm

    def body(x0, x1, x2, w, h, stats, a1, a2):
        i = pl.program_id(0)

        @pl.when(i == 0)
        def _():
            a1[...] = jnp.zeros_like(a1)
            a2[...] = jnp.zeros_like(a2)

        w_ = w[...]
        x0v = x0[...]
        x2c = 2.0 * x2[...] - x0v
        hv = (jnp.dot(x0v, w_[0:Fin, :], preferred_element_type=jnp.float32)
              + jnp.dot(x1[...], w_[Fin:2 * Fin, :],
                        preferred_element_type=jnp.float32)
              + jnp.dot(x2c, w_[2 * Fin:3 * Fin, :],
                        preferred_element_type=jnp.float32))
        h[...] = hv
        s = jnp.sum(hv, axis=0, keepdims=True)
        ss = jnp.sum(hv * hv, axis=0, keepdims=True)
        a1[...] += jnp.broadcast_to(s, (8, Fout))
        a2[...] += jnp.broadcast_to(ss, (8, Fout))

        @pl.when(i == grid - 1)
        def _():
            stats[0:8, :] = a1[...]
            stats[8:16, :] = a2[...]

    return pl.pallas_call(
        body,
        grid=(grid,),
        in_specs=[
            pl.BlockSpec((bm, Fin), lambda i: (i, 0)),
            pl.BlockSpec((bm, Fin), lambda i: (i, 0)),
            pl.BlockSpec((bm, Fin), lambda i: (i, 0)),
            pl.BlockSpec((3 * Fin, Fout), lambda i: (0, 0)),
        ],
        out_specs=[
            pl.BlockSpec((bm, Fout), lambda i: (i, 0)),
            pl.BlockSpec((16, Fout), lambda i: (0, 0)),
        ],
        out_shape=[
            jax.ShapeDtypeStruct((N, Fout), jnp.float32),
            jax.ShapeDtypeStruct((16, Fout), jnp.float32),
        ],
        scratch_shapes=[
            pltpu.VMEM((8, Fout), jnp.float32),
            pltpu.VMEM((8, Fout), jnp.float32),
        ],
    )


def _bn_coeffs(stats, g, b, N, F):
    mean = stats[0:1, :] * (1.0 / N)
    var = stats[8:9, :] * (1.0 / N) - mean * mean
    sc = g * lax.rsqrt(var + EPS)
    sh = b - mean * sc
    return sc, sh


def _mk_bn(N, F, bm):
    grid = N // bm

    def body(h, stats, g, b, y):
        sc, sh = _bn_coeffs(stats[...], g[...], b[...], N, F)
        y[...] = jnp.maximum(h[...] * sc + sh, 0.0)

    return pl.pallas_call(
        body,
        grid=(grid,),
        in_specs=[
            pl.BlockSpec((bm, F), lambda i: (i, 0)),
            pl.BlockSpec((16, F), lambda i: (0, 0)),
            pl.BlockSpec((1, F), lambda i: (0, 0)),
            pl.BlockSpec((1, F), lambda i: (0, 0)),
        ],
        out_specs=pl.BlockSpec((bm, F), lambda i: (i, 0)),
        out_shape=jax.ShapeDtypeStruct((N, F), jnp.float32),
    )


def _mk_bn_pool(N, F, bm4):
    N4 = N // 4
    grid = N4 // bm4

    def body(h3, stats, g, b, y3, p):
        sc, sh = _bn_coeffs(stats[...], g[...], b[...], N, F)
        y = jnp.maximum(h3[...] * sc.reshape(1, 1, F) + sh.reshape(1, 1, F),
                        0.0)
        y3[...] = y
        p[...] = jnp.max(y, axis=1)

    return pl.pallas_call(
        body,
        grid=(grid,),
        in_specs=[
            pl.BlockSpec((bm4, 4, F), lambda i: (i, 0, 0)),
            pl.BlockSpec((16, F), lambda i: (0, 0)),
            pl.BlockSpec((1, F), lambda i: (0, 0)),
            pl.BlockSpec((1, F), lambda i: (0, 0)),
        ],
        out_specs=[
            pl.BlockSpec((bm4, 4, F), lambda i: (i, 0, 0)),
            pl.BlockSpec((bm4, F), lambda i: (i, 0)),
        ],
        out_shape=[
            jax.ShapeDtypeStruct((N4, 4, F), jnp.float32),
            jax.ShapeDtypeStruct((N4, F), jnp.float32),
        ],
    )


def _fold_weights(W, Fin):
    W0, W1, W2 = W[0::3], W[1::3], W[2::3]
    return jnp.concatenate([W0, W1, W2], axis=0)


def _conv_bn(X, cols, vals, W, gamma, beta, V, Fin, Fout, pool):
    N = 2 * V
    mvk = _mk_matvec(V, Fin)
    X1 = mvk(X, cols, vals)
    X2 = mvk(X1, cols, vals)
    Wc = _fold_weights(W, Fin)
    bm = {98304: 8192, 24576: 4096, 6144: 2048}[N]
    outs = _mk_conv_fused(N, Fin, Fout, bm, pool)(
        X, X1, X2, Wc, gamma.reshape(1, Fout), beta.reshape(1, Fout))
    if pool:
        return outs[0], outs[1]
    return outs[0], None


def kernel(x, rows0, cols0, vals0, rows1, cols1, vals1, rows2, cols2, vals2,
           W1a, g1a, b1a, W1b, g1b, b1b, W2, g2, b2, W3, g3, b3):
    V0, V1, V2 = 49152, 12288, 3072
    X0 = x.reshape(2 * V0, 16)
    a, _ = _conv_bn(X0, cols0, vals0, W1a, g1a, b1a, V0, 16, 32, False)
    out1, p1 = _conv_bn(a, cols0, vals0, W1b, g1b, b1b, V0, 32, 64, True)
    out2, p2 = _conv_bn(p1, cols1, vals1, W2, g2, b2, V1, 64, 128, True)
    out3, _ = _conv_bn(p2, cols2, vals2, W3, g3, b3, V2, 128, 256, False)
    return (out3.reshape(2, V2, 256), out2.reshape(2, V1, 128),
            out1.reshape(2, V0, 64))


# speedup vs baseline: 265.5677x; 1.2341x over previous
import functools

import jax
import jax.numpy as jnp
from jax import lax
from jax.experimental import pallas as pl
from jax.experimental.pallas import tpu as pltpu
from jax.experimental.pallas import tpu_sc as plsc

NC, NS = 2, 16
NW = NC * NS
DEG = 8
EPS = 1e-5


def _mk_matvec(V, D2):
    rpt = V // NW
    C = min(rpt, max(16, 4096 // D2))
    while rpt % C:
        C //= 2
    nch = rpt // C
    ng = (DEG * C) // 128
    unroll = 4 if D2 <= 128 else 2
    mesh = plsc.VectorSubcoreMesh(core_axis_name="c", subcore_axis_name="s")

    @functools.partial(
        pl.kernel, mesh=mesh,
        compiler_params=pltpu.CompilerParams(use_tc_tiling_on_sc=False),
        out_type=jax.ShapeDtypeStruct((V, D2), jnp.float32),
        scratch_types=[
            pltpu.VMEM((2, ng, 128), jnp.int32),
            pltpu.VMEM((2, DEG * C), jnp.float32),
            pltpu.VMEM((2, DEG * C, D2), jnp.float32),
            pltpu.VMEM((C, D2), jnp.float32),
            pltpu.SemaphoreType.DMA,
        ],
    )
    def mv(x_hbm, cols2_hbm, vals_hbm, out_hbm, colbuf, valbuf, G, Y, sem):
        wid = lax.axis_index("s") * NC + lax.axis_index("c")
        row0 = wid * rpt

        def stage(ci, buf):
            vb = row0 + ci * C
            pltpu.sync_copy(cols2_hbm.at[pl.ds(vb * DEG // 128, ng)],
                            colbuf.at[buf])
            pltpu.sync_copy(vals_hbm.at[pl.ds(vb * DEG, DEG * C)],
                            valbuf.at[buf])
            for g in range(ng):
                pltpu.async_copy(
                    x_hbm.at[colbuf.at[buf].at[g]],
                    G.at[buf].at[pl.ds(g * 128, 128)], sem)

        def compute(ci, buf):
            pltpu.make_async_copy(
                x_hbm.at[pl.ds(0, DEG * C)], G.at[buf], sem).wait()

            def vert2(vi, c2):
                e0 = vi * 16
                vv = valbuf[buf, pl.ds(e0, 16)]
                for h in range(2):
                    base = e0 + h * DEG
                    for j in range(D2 // 16):
                        sl = pl.ds(j * 16, 16)
                        acc = G[buf, base, sl] * vv[h * DEG]
                        for d in range(1, DEG):
                            acc = acc + G[buf, base + d, sl] * vv[h * DEG + d]
                        Y[2 * vi + h, sl] = acc
                return c2

            lax.fori_loop(0, C // 2, vert2, 0, unroll=unroll)
            pltpu.sync_copy(Y, out_hbm.at[pl.ds(row0 + ci * C, C)])

        stage(0, 0)

        def pair(p, carry):
            n0 = 2 * p
            for off in range(2):
                n = n0 + off
                buf = off

                @pl.when(n + 1 < nch)
                def _():
                    stage(n + 1, 1 - buf)

                compute(n, buf)
            return carry

        lax.fori_loop(0, nch // 2, pair, 0)

    return mv


def _mk_conv_fused(V, Fin2, Fout2, bm, pool):
    nb = V // bm
    F = Fout2 // 2

    def body(x0, x1, x2, w, g, b, *rest):
        if pool:
            y, p, a1, a2 = rest
        else:
            (y, a1, a2) = rest
        ph = pl.program_id(0)
        i = pl.program_id(1)

        @pl.when(jnp.logical_and(ph == 0, i == 0))
        def _():
            a1[...] = jnp.zeros_like(a1)
            a2[...] = jnp.zeros_like(a2)

        w_ = w[...]
        x0v = x0[...]
        x2c = 2.0 * x2[...] - x0v
        hv = (jnp.dot(x0v, w_[0:Fin2, :], preferred_element_type=jnp.float32)
              + jnp.dot(x1[...], w_[Fin2:2 * Fin2, :],
                        preferred_element_type=jnp.float32)
              + jnp.dot(x2c, w_[2 * Fin2:3 * Fin2, :],
                        preferred_element_type=jnp.float32))

        @pl.when(ph == 0)
        def _():
            s = jnp.sum(hv, axis=0, keepdims=True)
            ss = jnp.sum(hv * hv, axis=0, keepdims=True)
            a1[...] += jnp.broadcast_to(s, (8, Fout2))
            a2[...] += jnp.broadcast_to(ss, (8, Fout2))

        @pl.when(ph == 1)
        def _():
            mh = a1[0:1, :] * (1.0 / V)
            eh = a2[0:1, :] * (1.0 / V)
            mhs = jnp.concatenate([mh[:, F:], mh[:, :F]], axis=1)
            ehs = jnp.concatenate([eh[:, F:], eh[:, :F]], axis=1)
            mean = 0.5 * (mh + mhs)
            var = 0.5 * (eh + ehs) - mean * mean
            sc = g[...] * lax.rsqrt(var + EPS)
            sh = b[...] - mean * sc
            yv = jnp.maximum(hv * sc + sh, 0.0)
            y[...] = yv
            if pool:
                p[...] = yv.reshape(bm // 4, 4, Fout2).max(axis=1)

    out_specs = [pl.BlockSpec((bm, Fout2), lambda ph, i: (ph * i, 0))]
    out_shape = [jax.ShapeDtypeStruct((V, Fout2), jnp.float32)]
    if pool:
        out_specs.append(
            pl.BlockSpec((bm // 4, Fout2), lambda ph, i: (ph * i, 0)))
        out_shape.append(jax.ShapeDtypeStruct((V // 4, Fout2), jnp.float32))

    return pl.pallas_call(
        body,
        grid=(2, nb),
        in_specs=[
            pl.BlockSpec((bm, Fin2), lambda ph, i: (i, 0)),
            pl.BlockSpec((bm, Fin2), lambda ph, i: (i, 0)),
            pl.BlockSpec((bm, Fin2), lambda ph, i: (i, 0)),
            pl.BlockSpec((3 * Fin2, Fout2), lambda ph, i: (0, 0)),
            pl.BlockSpec((1, Fout2), lambda ph, i: (0, 0)),
            pl.BlockSpec((1, Fout2), lambda ph, i: (0, 0)),
        ],
        out_specs=out_specs,
        out_shape=out_shape,
        scratch_shapes=[
            pltpu.VMEM((8, Fout2), jnp.float32),
            pltpu.VMEM((8, Fout2), jnp.float32),
        ],
    )


def _prep_weights(W, Fin, Fout):
    Z = jnp.zeros((Fin, Fout), W.dtype)
    blocks = []
    for k in range(3):
        Wk = W[k::3]
        blocks.append(jnp.concatenate(
            [jnp.concatenate([Wk, Z], axis=1),
             jnp.concatenate([Z, Wk], axis=1)], axis=0))
    return jnp.concatenate(blocks, axis=0)


def _conv_bn(X, cols2, vals, W, gamma, beta, V, Fin, Fout, pool):
    mvk = _mk_matvec(V, 2 * Fin)
    X1 = mvk(X, cols2, vals)
    X2 = mvk(X1, cols2, vals)
    Wd = _prep_weights(W, Fin, Fout)
    g2 = jnp.concatenate([gamma, gamma]).reshape(1, 2 * Fout)
    b2 = jnp.concatenate([beta, beta]).reshape(1, 2 * Fout)
    bm = {49152: 8192, 12288: 4096, 3072: 1024}[V]
    outs = _mk_conv_fused(V, 2 * Fin, 2 * Fout, bm, pool)(
        X, X1, X2, Wd, g2, b2)
    if pool:
        return outs[0], outs[1]
    return outs[0], None


def kernel(x, rows0, cols0, vals0, rows1, cols1, vals1, rows2, cols2, vals2,
           W1a, g1a, b1a, W1b, g1b, b1b, W2, g2, b2, W3, g3, b3):
    V0, V1, V2 = 49152, 12288, 3072
    X0 = x.transpose(1, 0, 2).reshape(V0, 32)
    c0, c1, c2 = (cols0.reshape(-1, 128), cols1.reshape(-1, 128),
                  cols2.reshape(-1, 128))
    a, _ = _conv_bn(X0, c0, vals0, W1a, g1a, b1a, V0, 16, 32, False)
    out1, p1 = _conv_bn(a, c0, vals0, W1b, g1b, b1b, V0, 32, 64, True)
    out2, p2 = _conv_bn(p1, c1, vals1, W2, g2, b2, V1, 64, 128, True)
    out3, _ = _conv_bn(p2, c2, vals2, W3, g3, b3, V2, 128, 256, False)
    return (out3.reshape(V2, 2, 256).transpose(1, 0, 2),
            out2.reshape(V1, 2, 128).transpose(1, 0, 2),
            out1.reshape(V0, 2, 64).transpose(1, 0, 2))


# speedup vs baseline: 275.0377x; 1.0357x over previous
import functools

import jax
import jax.numpy as jnp
from jax import lax
from jax.experimental import pallas as pl
from jax.experimental.pallas import tpu as pltpu
from jax.experimental.pallas import tpu_sc as plsc

NC, NS = 2, 16
NW = NC * NS
DEG = 8
EPS = 1e-5


def _mk_matvec(V, D2):
    rpt = V // NW
    C = min(rpt, max(16, 4096 // D2))
    while rpt % C:
        C //= 2
    nch = rpt // C
    ng = (DEG * C) // 128
    unroll = 4 if D2 <= 128 else 2
    mesh = plsc.VectorSubcoreMesh(core_axis_name="c", subcore_axis_name="s")

    @functools.partial(
        pl.kernel, mesh=mesh,
        compiler_params=pltpu.CompilerParams(use_tc_tiling_on_sc=False),
        out_type=jax.ShapeDtypeStruct((V, D2), jnp.float32),
        scratch_types=[
            pltpu.VMEM((2, ng, 128), jnp.int32),
            pltpu.VMEM((2, DEG * C), jnp.float32),
            pltpu.VMEM((2, DEG * C, D2), jnp.float32),
            pltpu.VMEM((C, D2), jnp.float32),
            pltpu.SemaphoreType.DMA,
        ],
    )
    def mv(x_hbm, cols2_hbm, vals_hbm, out_hbm, colbuf, valbuf, G, Y, sem):
        wid = lax.axis_index("s") * NC + lax.axis_index("c")
        row0 = wid * rpt

        def stage(ci, buf):
            vb = row0 + ci * C
            pltpu.sync_copy(cols2_hbm.at[pl.ds(vb * DEG // 128, ng)],
                            colbuf.at[buf])
            pltpu.sync_copy(vals_hbm.at[pl.ds(vb * DEG, DEG * C)],
                            valbuf.at[buf])
            for g in range(ng):
                pltpu.async_copy(
                    x_hbm.at[colbuf.at[buf].at[g]],
                    G.at[buf].at[pl.ds(g * 128, 128)], sem)

        def compute(ci, buf):
            pltpu.make_async_copy(
                x_hbm.at[pl.ds(0, DEG * C)], G.at[buf], sem).wait()

            def vert2(vi, c2):
                e0 = vi * 16
                vv = valbuf[buf, pl.ds(e0, 16)]
                for h in range(2):
                    base = e0 + h * DEG
                    for j in range(D2 // 16):
                        sl = pl.ds(j * 16, 16)
                        p = [G[buf, base + d, sl] * vv[h * DEG + d]
                             for d in range(DEG)]
                        s = ((p[0] + p[1]) + (p[2] + p[3])) + \
                            ((p[4] + p[5]) + (p[6] + p[7]))
                        Y[2 * vi + h, sl] = s
                return c2

            lax.fori_loop(0, C // 2, vert2, 0, unroll=unroll)
            pltpu.sync_copy(Y, out_hbm.at[pl.ds(row0 + ci * C, C)])

        stage(0, 0)

        def pair(p, carry):
            n0 = 2 * p
            for off in range(2):
                n = n0 + off
                buf = off

                @pl.when(n + 1 < nch)
                def _():
                    stage(n + 1, 1 - buf)

                compute(n, buf)
            return carry

        lax.fori_loop(0, nch // 2, pair, 0)

    return mv


def _mk_conv_fused(V, Fin2, Fout2, bm, pool):
    nb = V // bm
    F = Fout2 // 2

    def body(x0, x1, x2, w, g, b, *rest):
        if pool:
            y, p, a1, a2 = rest
        else:
            (y, a1, a2) = rest
        ph = pl.program_id(0)
        i = pl.program_id(1)

        @pl.when(jnp.logical_and(ph == 0, i == 0))
        def _():
            a1[...] = jnp.zeros_like(a1)
            a2[...] = jnp.zeros_like(a2)

        w_ = w[...]
        x0v = x0[...]
        x2c = 2.0 * x2[...] - x0v
        hv = (jnp.dot(x0v, w_[0:Fin2, :], preferred_element_type=jnp.float32)
              + jnp.dot(x1[...], w_[Fin2:2 * Fin2, :],
                        preferred_element_type=jnp.float32)
              + jnp.dot(x2c, w_[2 * Fin2:3 * Fin2, :],
                        preferred_element_type=jnp.float32))

        @pl.when(ph == 0)
        def _():
            s = jnp.sum(hv, axis=0, keepdims=True)
            ss = jnp.sum(hv * hv, axis=0, keepdims=True)
            a1[...] += jnp.broadcast_to(s, (8, Fout2))
            a2[...] += jnp.broadcast_to(ss, (8, Fout2))

        @pl.when(ph == 1)
        def _():
            mh = a1[0:1, :] * (1.0 / V)
            eh = a2[0:1, :] * (1.0 / V)
            mhs = jnp.concatenate([mh[:, F:], mh[:, :F]], axis=1)
            ehs = jnp.concatenate([eh[:, F:], eh[:, :F]], axis=1)
            mean = 0.5 * (mh + mhs)
            var = 0.5 * (eh + ehs) - mean * mean
            sc = g[...] * lax.rsqrt(var + EPS)
            sh = b[...] - mean * sc
            yv = jnp.maximum(hv * sc + sh, 0.0)
            y[...] = yv
            if pool:
                p[...] = yv.reshape(bm // 4, 4, Fout2).max(axis=1)

    out_specs = [pl.BlockSpec((bm, Fout2), lambda ph, i: (ph * i, 0))]
    out_shape = [jax.ShapeDtypeStruct((V, Fout2), jnp.float32)]
    if pool:
        out_specs.append(
            pl.BlockSpec((bm // 4, Fout2), lambda ph, i: (ph * i, 0)))
        out_shape.append(jax.ShapeDtypeStruct((V // 4, Fout2), jnp.float32))

    return pl.pallas_call(
        body,
        grid=(2, nb),
        in_specs=[
            pl.BlockSpec((bm, Fin2), lambda ph, i: (i, 0)),
            pl.BlockSpec((bm, Fin2), lambda ph, i: (i, 0)),
            pl.BlockSpec((bm, Fin2), lambda ph, i: (i, 0)),
            pl.BlockSpec((3 * Fin2, Fout2), lambda ph, i: (0, 0)),
            pl.BlockSpec((1, Fout2), lambda ph, i: (0, 0)),
            pl.BlockSpec((1, Fout2), lambda ph, i: (0, 0)),
        ],
        out_specs=out_specs,
        out_shape=out_shape,
        scratch_shapes=[
            pltpu.VMEM((8, Fout2), jnp.float32),
            pltpu.VMEM((8, Fout2), jnp.float32),
        ],
    )


def _prep_weights(W, Fin, Fout):
    Z = jnp.zeros((Fin, Fout), W.dtype)
    blocks = []
    for k in range(3):
        Wk = W[k::3]
        blocks.append(jnp.concatenate(
            [jnp.concatenate([Wk, Z], axis=1),
             jnp.concatenate([Z, Wk], axis=1)], axis=0))
    return jnp.concatenate(blocks, axis=0)


def _conv_bn(X, cols2, vals, W, gamma, beta, V, Fin, Fout, pool):
    mvk = _mk_matvec(V, 2 * Fin)
    X1 = mvk(X, cols2, vals)
    X2 = mvk(X1, cols2, vals)
    Wd = _prep_weights(W, Fin, Fout)
    g2 = jnp.concatenate([gamma, gamma]).reshape(1, 2 * Fout)
    b2 = jnp.concatenate([beta, beta]).reshape(1, 2 * Fout)
    bm = {49152: 8192, 12288: 4096, 3072: 1024}[V]
    outs = _mk_conv_fused(V, 2 * Fin, 2 * Fout, bm, pool)(
        X, X1, X2, Wd, g2, b2)
    if pool:
        return outs[0], outs[1]
    return outs[0], None


def kernel(x, rows0, cols0, vals0, rows1, cols1, vals1, rows2, cols2, vals2,
           W1a, g1a, b1a, W1b, g1b, b1b, W2, g2, b2, W3, g3, b3):
    V0, V1, V2 = 49152, 12288, 3072
    X0 = x.transpose(1, 0, 2).reshape(V0, 32)
    c0, c1, c2 = (cols0.reshape(-1, 128), cols1.reshape(-1, 128),
                  cols2.reshape(-1, 128))
    a, _ = _conv_bn(X0, c0, vals0, W1a, g1a, b1a, V0, 16, 32, False)
    out1, p1 = _conv_bn(a, c0, vals0, W1b, g1b, b1b, V0, 32, 64, True)
    out2, p2 = _conv_bn(p1, c1, vals1, W2, g2, b2, V1, 64, 128, True)
    out3, _ = _conv_bn(p2, c2, vals2, W3, g3, b3, V2, 128, 256, False)
    return (out3.reshape(V2, 2, 256).transpose(1, 0, 2),
            out2.reshape(V1, 2, 128).transpose(1, 0, 2),
            out1.reshape(V0, 2, 64).transpose(1, 0, 2))


# speedup vs baseline: 286.3402x; 1.0411x over previous
import functools

import jax
import jax.numpy as jnp
from jax import lax
from jax.experimental import pallas as pl
from jax.experimental.pallas import tpu as pltpu
from jax.experimental.pallas import tpu_sc as plsc

NC, NS = 2, 16
NW = NC * NS
DEG = 8
EPS = 1e-5


def _mk_matvec(V, D2):
    rpt = V // NW
    C = min(rpt, max(16, (196608 // (DEG * D2 * 4)) // 16 * 16))
    while rpt % C:
        C //= 2
    nch = rpt // C
    ng = (DEG * C) // 128
    unroll = 4 if D2 <= 128 else 2
    mesh = plsc.VectorSubcoreMesh(core_axis_name="c", subcore_axis_name="s")

    @functools.partial(
        pl.kernel, mesh=mesh,
        compiler_params=pltpu.CompilerParams(use_tc_tiling_on_sc=False),
        out_type=jax.ShapeDtypeStruct((V, D2), jnp.float32),
        scratch_types=[
            pltpu.VMEM((2, ng, 128), jnp.int32),
            pltpu.VMEM((2, DEG * C), jnp.float32),
            pltpu.VMEM((2, DEG * C, D2), jnp.float32),
            pltpu.VMEM((C, D2), jnp.float32),
            pltpu.SemaphoreType.DMA,
        ],
    )
    def mv(x_hbm, cols2_hbm, vals_hbm, out_hbm, colbuf, valbuf, G, Y, sem):
        wid = lax.axis_index("s") * NC + lax.axis_index("c")
        row0 = wid * rpt

        def stage(ci, buf):
            vb = row0 + ci * C
            pltpu.sync_copy(cols2_hbm.at[pl.ds(vb * DEG // 128, ng)],
                            colbuf.at[buf])
            pltpu.sync_copy(vals_hbm.at[pl.ds(vb * DEG, DEG * C)],
                            valbuf.at[buf])
            for g in range(ng):
                pltpu.async_copy(
                    x_hbm.at[colbuf.at[buf].at[g]],
                    G.at[buf].at[pl.ds(g * 128, 128)], sem)

        def compute(ci, buf):
            pltpu.make_async_copy(
                x_hbm.at[pl.ds(0, DEG * C)], G.at[buf], sem).wait()

            def vert2(vi, c2):
                e0 = vi * 16
                vv = valbuf[buf, pl.ds(e0, 16)]
                for h in range(2):
                    base = e0 + h * DEG
                    for j in range(D2 // 16):
                        sl = pl.ds(j * 16, 16)
                        p = [G[buf, base + d, sl] * vv[h * DEG + d]
                             for d in range(DEG)]
                        s = ((p[0] + p[1]) + (p[2] + p[3])) + \
                            ((p[4] + p[5]) + (p[6] + p[7]))
                        Y[2 * vi + h, sl] = s
                return c2

            lax.fori_loop(0, C // 2, vert2, 0, unroll=unroll)
            pltpu.sync_copy(Y, out_hbm.at[pl.ds(row0 + ci * C, C)])

        stage(0, 0)

        def pair(p, carry):
            n0 = 2 * p
            for off in range(2):
                n = n0 + off
                buf = off

                @pl.when(n + 1 < nch)
                def _():
                    stage(n + 1, 1 - buf)

                compute(n, buf)
            return carry

        lax.fori_loop(0, nch // 2, pair, 0)

    return mv


def _mk_conv_fused(V, Fin2, Fout2, bm, pool):
    nb = V // bm
    F = Fout2 // 2

    def body(x0, x1, x2, w, g, b, *rest):
        if pool:
            y, p, a1, a2 = rest
        else:
            (y, a1, a2) = rest
        ph = pl.program_id(0)
        i = pl.program_id(1)

        @pl.when(jnp.logical_and(ph == 0, i == 0))
        def _():
            a1[...] = jnp.zeros_like(a1)
            a2[...] = jnp.zeros_like(a2)

        w_ = w[...]
        x0v = x0[...]
        x2c = 2.0 * x2[...] - x0v
        hv = (jnp.dot(x0v, w_[0:Fin2, :], preferred_element_type=jnp.float32)
              + jnp.dot(x1[...], w_[Fin2:2 * Fin2, :],
                        preferred_element_type=jnp.float32)
              + jnp.dot(x2c, w_[2 * Fin2:3 * Fin2, :],
                        preferred_element_type=jnp.float32))

        @pl.when(ph == 0)
        def _():
            s = jnp.sum(hv, axis=0, keepdims=True)
            ss = jnp.sum(hv * hv, axis=0, keepdims=True)
            a1[...] += jnp.broadcast_to(s, (8, Fout2))
            a2[...] += jnp.broadcast_to(ss, (8, Fout2))

        @pl.when(ph == 1)
        def _():
            mh = a1[0:1, :] * (1.0 / V)
            eh = a2[0:1, :] * (1.0 / V)
            mhs = jnp.concatenate([mh[:, F:], mh[:, :F]], axis=1)
            ehs = jnp.concatenate([eh[:, F:], eh[:, :F]], axis=1)
            mean = 0.5 * (mh + mhs)
            var = 0.5 * (eh + ehs) - mean * mean
            sc = g[...] * lax.rsqrt(var + EPS)
            sh = b[...] - mean * sc
            yv = jnp.maximum(hv * sc + sh, 0.0)
            y[...] = yv
            if pool:
                p[...] = yv.reshape(bm // 4, 4, Fout2).max(axis=1)

    out_specs = [pl.BlockSpec((bm, Fout2), lambda ph, i: (ph * i, 0))]
    out_shape = [jax.ShapeDtypeStruct((V, Fout2), jnp.float32)]
    if pool:
        out_specs.append(
            pl.BlockSpec((bm // 4, Fout2), lambda ph, i: (ph * i, 0)))
        out_shape.append(jax.ShapeDtypeStruct((V // 4, Fout2), jnp.float32))

    return pl.pallas_call(
        body,
        grid=(2, nb),
        in_specs=[
            pl.BlockSpec((bm, Fin2), lambda ph, i: (i, 0)),
            pl.BlockSpec((bm, Fin2), lambda ph, i: (i, 0)),
            pl.BlockSpec((bm, Fin2), lambda ph, i: (i, 0)),
            pl.BlockSpec((3 * Fin2, Fout2), lambda ph, i: (0, 0)),
            pl.BlockSpec((1, Fout2), lambda ph, i: (0, 0)),
            pl.BlockSpec((1, Fout2), lambda ph, i: (0, 0)),
        ],
        out_specs=out_specs,
        out_shape=out_shape,
        scratch_shapes=[
            pltpu.VMEM((8, Fout2), jnp.float32),
            pltpu.VMEM((8, Fout2), jnp.float32),
        ],
    )


def _prep_weights(W, Fin, Fout):
    Z = jnp.zeros((Fin, Fout), W.dtype)
    blocks = []
    for k in range(3):
        Wk = W[k::3]
        blocks.append(jnp.concatenate(
            [jnp.concatenate([Wk, Z], axis=1),
             jnp.concatenate([Z, Wk], axis=1)], axis=0))
    return jnp.concatenate(blocks, axis=0)


def _conv_bn(X, cols2, vals, W, gamma, beta, V, Fin, Fout, pool):
    mvk = _mk_matvec(V, 2 * Fin)
    X1 = mvk(X, cols2, vals)
    X2 = mvk(X1, cols2, vals)
    Wd = _prep_weights(W, Fin, Fout)
    g2 = jnp.concatenate([gamma, gamma]).reshape(1, 2 * Fout)
    b2 = jnp.concatenate([beta, beta]).reshape(1, 2 * Fout)
    bm = {49152: 8192, 12288: 4096, 3072: 1024}[V]
    outs = _mk_conv_fused(V, 2 * Fin, 2 * Fout, bm, pool)(
        X, X1, X2, Wd, g2, b2)
    if pool:
        return outs[0], outs[1]
    return outs[0], None


def kernel(x, rows0, cols0, vals0, rows1, cols1, vals1, rows2, cols2, vals2,
           W1a, g1a, b1a, W1b, g1b, b1b, W2, g2, b2, W3, g3, b3):
    V0, V1, V2 = 49152, 12288, 3072
    X0 = x.transpose(1, 0, 2).reshape(V0, 32)
    c0, c1, c2 = (cols0.reshape(-1, 128), cols1.reshape(-1, 128),
                  cols2.reshape(-1, 128))
    a, _ = _conv_bn(X0, c0, vals0, W1a, g1a, b1a, V0, 16, 32, False)
    out1, p1 = _conv_bn(a, c0, vals0, W1b, g1b, b1b, V0, 32, 64, True)
    out2, p2 = _conv_bn(p1, c1, vals1, W2, g2, b2, V1, 64, 128, True)
    out3, _ = _conv_bn(p2, c2, vals2, W3, g3, b3, V2, 128, 256, False)
    return (out3.reshape(V2, 2, 256).transpose(1, 0, 2),
            out2.reshape(V1, 2, 128).transpose(1, 0, 2),
            out1.reshape(V0, 2, 64).transpose(1, 0, 2))


# speedup vs baseline: 303.6115x; 1.0603x over previous
import functools

import jax
import jax.numpy as jnp
from jax import lax
from jax.experimental import pallas as pl
from jax.experimental.pallas import tpu as pltpu
from jax.experimental.pallas import tpu_sc as plsc

NC, NS = 2, 16
NW = NC * NS
DEG = 8
EPS = 1e-5


def _mk_matvec(V, D2):
    rpt = V // NW
    C = min(rpt, max(16, (196608 // (DEG * D2 * 4)) // 16 * 16))
    while rpt % C:
        C //= 2
    nch = rpt // C
    ng = (DEG * C) // 128
    unroll = 4 if D2 <= 128 else 2
    mesh = plsc.VectorSubcoreMesh(core_axis_name="c", subcore_axis_name="s")

    @functools.partial(
        pl.kernel, mesh=mesh,
        compiler_params=pltpu.CompilerParams(use_tc_tiling_on_sc=False),
        out_type=jax.ShapeDtypeStruct((V, D2), jnp.float32),
        scratch_types=[
            pltpu.VMEM((2, ng, 128), jnp.int32),
            pltpu.VMEM((2, DEG * C), jnp.float32),
            pltpu.VMEM((2, DEG * C, D2), jnp.float32),
            pltpu.VMEM((2, C, D2), jnp.float32),
            pltpu.SemaphoreType.DMA,
            pltpu.SemaphoreType.DMA,
            pltpu.SemaphoreType.DMA,
            pltpu.SemaphoreType.DMA,
            pltpu.SemaphoreType.DMA,
        ],
    )
    def mv(x_hbm, cols2_hbm, vals_hbm, out_hbm, colbuf, valbuf, G, Y,
           semG, semC0, semC1, semY0, semY1):
        wid = lax.axis_index("s") * NC + lax.axis_index("c")
        row0 = wid * rpt
        semC = (semC0, semC1)
        semY = (semY0, semY1)

        def loadcv(ci, buf):
            vb = row0 + ci * C
            pltpu.async_copy(cols2_hbm.at[pl.ds(vb * DEG // 128, ng)],
                             colbuf.at[buf], semC[buf])
            pltpu.async_copy(vals_hbm.at[pl.ds(vb * DEG, DEG * C)],
                             valbuf.at[buf], semC[buf])

        def waitcv(buf):
            pltpu.make_async_copy(cols2_hbm.at[pl.ds(0, ng)],
                                  colbuf.at[buf], semC[buf]).wait()
            pltpu.make_async_copy(vals_hbm.at[pl.ds(0, DEG * C)],
                                  valbuf.at[buf], semC[buf]).wait()

        def fire_g(buf):
            for g in range(ng):
                pltpu.async_copy(
                    x_hbm.at[colbuf.at[buf].at[g]],
                    G.at[buf].at[pl.ds(g * 128, 128)], semG)

        def drain_g(buf):
            pltpu.make_async_copy(
                x_hbm.at[pl.ds(0, DEG * C)], G.at[buf], semG).wait()

        def drain_y(buf):
            pltpu.make_async_copy(
                Y.at[buf], out_hbm.at[pl.ds(0, C)], semY[buf]).wait()

        def compute(ci, buf):
            def vert2(vi, c2):
                e0 = vi * 16
                vv = valbuf[buf, pl.ds(e0, 16)]
                for h in range(2):
                    base = e0 + h * DEG
                    for j in range(D2 // 16):
                        sl = pl.ds(j * 16, 16)
                        p = [G[buf, base + d, sl] * vv[h * DEG + d]
                             for d in range(DEG)]
                        s = ((p[0] + p[1]) + (p[2] + p[3])) + \
                            ((p[4] + p[5]) + (p[6] + p[7]))
                        Y[buf, 2 * vi + h, sl] = s
                return c2

            lax.fori_loop(0, C // 2, vert2, 0, unroll=unroll)
            pltpu.async_copy(Y.at[buf],
                             out_hbm.at[pl.ds(row0 + ci * C, C)], semY[buf])

        loadcv(0, 0)
        waitcv(0)
        fire_g(0)
        loadcv(1, 1)

        def pair(p, carry):
            n0 = 2 * p
            for off in range(2):
                n = n0 + off
                buf = off
                other = 1 - off

                @pl.when(n + 1 < nch)
                def _():
                    waitcv(other)
                    fire_g(other)

                drain_g(buf)

                @pl.when(n >= 2)
                def _():
                    drain_y(buf)

                compute(n, buf)

                @pl.when(n + 2 < nch)
                def _():
                    loadcv(n + 2, buf)
            return carry

        lax.fori_loop(0, nch // 2, pair, 0)
        drain_y(0)
        drain_y(1)

    return mv


def _mk_conv_fused(V, Fin2, Fout2, bm, pool):
    nb = V // bm
    F = Fout2 // 2

    def body(x0, x1, x2, w, g, b, *rest):
        if pool:
            y, p, a1, a2 = rest
        else:
            (y, a1, a2) = rest
        ph = pl.program_id(0)
        i = pl.program_id(1)

        @pl.when(jnp.logical_and(ph == 0, i == 0))
        def _():
            a1[...] = jnp.zeros_like(a1)
            a2[...] = jnp.zeros_like(a2)

        w_ = w[...]
        x0v = x0[...]
        x2c = 2.0 * x2[...] - x0v
        hv = (jnp.dot(x0v, w_[0:Fin2, :], preferred_element_type=jnp.float32)
              + jnp.dot(x1[...], w_[Fin2:2 * Fin2, :],
                        preferred_element_type=jnp.float32)
              + jnp.dot(x2c, w_[2 * Fin2:3 * Fin2, :],
                        preferred_element_type=jnp.float32))

        @pl.when(ph == 0)
        def _():
            s = jnp.sum(hv, axis=0, keepdims=True)
            ss = jnp.sum(hv * hv, axis=0, keepdims=True)
            a1[...] += jnp.broadcast_to(s, (8, Fout2))
            a2[...] += jnp.broadcast_to(ss, (8, Fout2))

        @pl.when(ph == 1)
        def _():
            mh = a1[0:1, :] * (1.0 / V)
            eh = a2[0:1, :] * (1.0 / V)
            mhs = jnp.concatenate([mh[:, F:], mh[:, :F]], axis=1)
            ehs = jnp.concatenate([eh[:, F:], eh[:, :F]], axis=1)
            mean = 0.5 * (mh + mhs)
            var = 0.5 * (eh + ehs) - mean * mean
            sc = g[...] * lax.rsqrt(var + EPS)
            sh = b[...] - mean * sc
            yv = jnp.maximum(hv * sc + sh, 0.0)
            y[...] = yv
            if pool:
                p[...] = yv.reshape(bm // 4, 4, Fout2).max(axis=1)

    out_specs = [pl.BlockSpec((bm, Fout2), lambda ph, i: (ph * i, 0))]
    out_shape = [jax.ShapeDtypeStruct((V, Fout2), jnp.float32)]
    if pool:
        out_specs.append(
            pl.BlockSpec((bm // 4, Fout2), lambda ph, i: (ph * i, 0)))
        out_shape.append(jax.ShapeDtypeStruct((V // 4, Fout2), jnp.float32))

    return pl.pallas_call(
        body,
        grid=(2, nb),
        in_specs=[
            pl.BlockSpec((bm, Fin2), lambda ph, i: (i, 0)),
            pl.BlockSpec((bm, Fin2), lambda ph, i: (i, 0)),
            pl.BlockSpec((bm, Fin2), lambda ph, i: (i, 0)),
            pl.BlockSpec((3 * Fin2, Fout2), lambda ph, i: (0, 0)),
            pl.BlockSpec((1, Fout2), lambda ph, i: (0, 0)),
            pl.BlockSpec((1, Fout2), lambda ph, i: (0, 0)),
        ],
        out_specs=out_specs,
        out_shape=out_shape,
        scratch_shapes=[
            pltpu.VMEM((8, Fout2), jnp.float32),
            pltpu.VMEM((8, Fout2), jnp.float32),
        ],
    )


def _prep_weights(W, Fin, Fout):
    Z = jnp.zeros((Fin, Fout), W.dtype)
    blocks = []
    for k in range(3):
        Wk = W[k::3]
        blocks.append(jnp.concatenate(
            [jnp.concatenate([Wk, Z], axis=1),
             jnp.concatenate([Z, Wk], axis=1)], axis=0))
    return jnp.concatenate(blocks, axis=0)


def _conv_bn(X, cols2, vals, W, gamma, beta, V, Fin, Fout, pool):
    mvk = _mk_matvec(V, 2 * Fin)
    X1 = mvk(X, cols2, vals)
    X2 = mvk(X1, cols2, vals)
    Wd = _prep_weights(W, Fin, Fout)
    g2 = jnp.concatenate([gamma, gamma]).reshape(1, 2 * Fout)
    b2 = jnp.concatenate([beta, beta]).reshape(1, 2 * Fout)
    bm = {49152: 8192, 12288: 4096, 3072: 1024}[V]
    outs = _mk_conv_fused(V, 2 * Fin, 2 * Fout, bm, pool)(
        X, X1, X2, Wd, g2, b2)
    if pool:
        return outs[0], outs[1]
    return outs[0], None


def kernel(x, rows0, cols0, vals0, rows1, cols1, vals1, rows2, cols2, vals2,
           W1a, g1a, b1a, W1b, g1b, b1b, W2, g2, b2, W3, g3, b3):
    V0, V1, V2 = 49152, 12288, 3072
    X0 = x.transpose(1, 0, 2).reshape(V0, 32)
    c0, c1, c2 = (cols0.reshape(-1, 128), cols1.reshape(-1, 128),
                  cols2.reshape(-1, 128))
    a, _ = _conv_bn(X0, c0, vals0, W1a, g1a, b1a, V0, 16, 32, False)
    out1, p1 = _conv_bn(a, c0, vals0, W1b, g1b, b1b, V0, 32, 64, True)
    out2, p2 = _conv_bn(p1, c1, vals1, W2, g2, b2, V1, 64, 128, True)
    out3, _ = _conv_bn(p2, c2, vals2, W3, g3, b3, V2, 128, 256, False)
    return (out3.reshape(V2, 2, 256).transpose(1, 0, 2),
            out2.reshape(V1, 2, 128).transpose(1, 0, 2),
            out1.reshape(V0, 2, 64).transpose(1, 0, 2))
